# baseline (speedup 1.0000x reference)
import functools

import jax
import jax.numpy as jnp
from jax import lax
from jax.experimental import pallas as pl
from jax.experimental.pallas import tpu as pltpu
from jax.experimental.pallas import tpu_sc as plsc

_EMB = 64
_SCALE = 8.0
_L = 16
_NC = 2
_NS = 16
_NW = _NC * _NS

_CHUNK = 512
_IDXW = 128
_NSTREAM = _CHUNK // _IDXW
_ROWS_PER_ITER = 4


@functools.lru_cache(maxsize=None)
def _make_kernel(n_idx, vocab):
    b_per_w = n_idx // _NW
    n_chunks = b_per_w // _CHUNK
    idx_rows_per_chunk = _CHUNK // _IDXW

    mesh = plsc.VectorSubcoreMesh(core_axis_name="c", subcore_axis_name="s")

    @functools.partial(
        pl.kernel,
        mesh=mesh,
        compiler_params=pltpu.CompilerParams(use_tc_tiling_on_sc=False),
        out_type=jax.ShapeDtypeStruct((n_idx, _EMB), jnp.float32),
        scratch_types=[
            pltpu.VMEM((idx_rows_per_chunk, _IDXW), jnp.int32),
            pltpu.VMEM((_CHUNK, _EMB), jnp.float32),
            pltpu.SemaphoreType.DMA,
        ],
    )
    def emb(x_hbm, table_hbm, out_hbm, idx_v, rows_v, sem):
        wid = lax.axis_index("s") * _NC + lax.axis_index("c")
        idx_row0 = wid * (b_per_w // _IDXW)
        out0 = wid * b_per_w

        def chunk_body(ci, carry):
            pltpu.sync_copy(
                x_hbm.at[pl.ds(idx_row0 + ci * idx_rows_per_chunk,
                               idx_rows_per_chunk)],
                idx_v)
            copies = [
                pltpu.async_copy(
                    table_hbm.at[idx_v.at[j]],
                    rows_v.at[pl.ds(j * _IDXW, _IDXW)],
                    sem)
                for j in range(_NSTREAM)
            ]
            for cp in copies:
                cp.wait()

            def scale_body(r, c2):
                for u in range(_ROWS_PER_ITER):
                    row = r * _ROWS_PER_ITER + u
                    for c in range(_EMB // _L):
                        sl = (row, pl.ds(c * _L, _L))
                        rows_v[sl] = rows_v[sl] * _SCALE
                return c2

            lax.fori_loop(0, _CHUNK // _ROWS_PER_ITER, scale_body, 0)
            pltpu.sync_copy(rows_v,
                            out_hbm.at[pl.ds(out0 + ci * _CHUNK, _CHUNK)])
            return carry

        lax.fori_loop(0, n_chunks, chunk_body, 0)

    return emb


def kernel(x, table):
    n_idx = x.size
    x2d = x.reshape(n_idx // _IDXW, _IDXW).astype(jnp.int32)
    out = _make_kernel(n_idx, table.shape[0])(x2d, table)
    return out.reshape(*x.shape, _EMB)


# speedup vs baseline: 1.0940x; 1.0940x over previous
import functools

import jax
import jax.numpy as jnp
from jax import lax
from jax.experimental import pallas as pl
from jax.experimental.pallas import tpu as pltpu
from jax.experimental.pallas import tpu_sc as plsc

_EMB = 64
_SCALE = 8.0
_L = 16
_NC = 2
_NS = 16
_NW = _NC * _NS

_IDXW = 128
_CHUNK = 256
_NSTREAM = _CHUNK // _IDXW
_NBUF = 4
_LOOKAHEAD = 2
_ROWS_PER_ITER = 8


@functools.lru_cache(maxsize=None)
def _make_kernel(n_idx):
    b_per_w = n_idx // _NW
    n_chunks = b_per_w // _CHUNK
    idx_rows_per_w = b_per_w // _IDXW

    mesh = plsc.VectorSubcoreMesh(core_axis_name="c", subcore_axis_name="s")

    @functools.partial(
        pl.kernel,
        mesh=mesh,
        compiler_params=pltpu.CompilerParams(use_tc_tiling_on_sc=False),
        out_type=jax.ShapeDtypeStruct((n_idx, _EMB), jnp.float32),
        scratch_types=[
            pltpu.VMEM((idx_rows_per_w, _IDXW), jnp.int32),
            pltpu.VMEM((_NBUF, _CHUNK, _EMB), jnp.float32),
        ] + [pltpu.SemaphoreType.DMA] * (2 * _NBUF),
    )
    def emb(x_hbm, table_hbm, out_hbm, idx_all, rows, *sems):
        gsem = sems[:_NBUF]
        osem = sems[_NBUF:]
        wid = lax.axis_index("s") * _NC + lax.axis_index("c")
        out0 = wid * b_per_w

        pltpu.sync_copy(x_hbm.at[pl.ds(wid * idx_rows_per_w, idx_rows_per_w)],
                        idx_all)

        def start_gather(ci, b):
            for j in range(_NSTREAM):
                pltpu.async_copy(
                    table_hbm.at[idx_all.at[ci * _NSTREAM + j]],
                    rows.at[b, pl.ds(j * _IDXW, _IDXW)],
                    gsem[b])

        def wait_gather(ci, b):
            for j in range(_NSTREAM):
                pltpu.make_async_copy(
                    table_hbm.at[idx_all.at[ci * _NSTREAM + j]],
                    rows.at[b, pl.ds(j * _IDXW, _IDXW)],
                    gsem[b]).wait()

        def start_out(ci, b):
            pltpu.async_copy(rows.at[b],
                             out_hbm.at[pl.ds(out0 + ci * _CHUNK, _CHUNK)],
                             osem[b])

        def wait_out(ci, b):
            pltpu.make_async_copy(rows.at[b],
                                  out_hbm.at[pl.ds(out0 + ci * _CHUNK, _CHUNK)],
                                  osem[b]).wait()

        def scale(b):
            def scale_body(r, carry):
                for u in range(_ROWS_PER_ITER):
                    row = r * _ROWS_PER_ITER + u
                    for c in range(_EMB // _L):
                        sl = (b, row, pl.ds(c * _L, _L))
                        rows[sl] = rows[sl] * _SCALE
                return carry
            lax.fori_loop(0, _CHUNK // _ROWS_PER_ITER, scale_body, 0,
                          unroll=False)

        def step(ci, b, refill, refill_waits=True):
            rc = ci + _LOOKAHEAD
            rb = (b + _LOOKAHEAD) % _NBUF
            if refill:
                if refill_waits:
                    wait_out(rc - _NBUF, rb)
                start_gather(rc, rb)
            wait_gather(ci, b)
            scale(b)
            start_out(ci, b)

        for ci in range(_LOOKAHEAD):
            start_gather(ci, ci % _NBUF)

        for ci in range(_NBUF):
            step(ci, ci % _NBUF, refill=True,
                 refill_waits=(ci + _LOOKAHEAD >= _NBUF))

        def group_body(g, carry):
            for b in range(_NBUF):
                step(g * _NBUF + b, b, refill=True)
            return carry

        lax.fori_loop(1, n_chunks // _NBUF - 1, group_body, 0, unroll=False)

        for ci in range(n_chunks - _NBUF, n_chunks):
            step(ci, ci % _NBUF, refill=(ci + _LOOKAHEAD < n_chunks))

        for ci in range(n_chunks - _NBUF, n_chunks):
            wait_out(ci, ci % _NBUF)

    return emb


def kernel(x, table):
    n_idx = x.size
    x2d = x.reshape(n_idx // _IDXW, _IDXW).astype(jnp.int32)
    out = _make_kernel(n_idx)(x2d, table)
    return out.reshape(*x.shape, _EMB)
